# baseline (speedup 1.0000x reference)
import jax
import jax.numpy as jnp
from jax.experimental import pallas as pl
from jax.experimental.pallas import tpu as pltpu

SUBLANE = 8


def _mlp_kernel(x_hbm, w1_ref, w2_ref, w3_ref, b_ref, o_ref,
                x_buf, in_sem, *, tb, nsteps):
    f_p = w1_ref.shape[1]
    out_p = w3_ref.shape[1]
    n_out = o_ref.shape[-1]

    i = pl.program_id(0)
    cur = jax.lax.rem(i, 2)
    nxt = jax.lax.rem(i + 1, 2)

    def start_copy(blk, slot):
        pltpu.make_async_copy(
            x_hbm.at[pl.ds(blk * tb, tb), 0, :], x_buf.at[slot],
            in_sem.at[slot]).start()

    @pl.when(i == 0)
    def _():
        start_copy(0, 0)

    @pl.when(i + 1 < nsteps)
    def _():
        start_copy(i + 1, nxt)

    pltpu.make_async_copy(
        x_hbm.at[pl.ds(0, tb), 0, :], x_buf.at[cur], in_sem.at[cur]).wait()

    x = x_buf[cur]
    h = jnp.dot(x, w1_ref[...], preferred_element_type=jnp.float32)
    h = jnp.maximum(h + b_ref[0:1, 0:f_p], 0.0)
    h = jnp.dot(h, w2_ref[...], preferred_element_type=jnp.float32)
    h = jnp.maximum(h + b_ref[1:2, 0:f_p], 0.0)
    a = jnp.dot(h, w3_ref[...], preferred_element_type=jnp.float32)
    o_ref[...] = (a + b_ref[2:3, 0:out_p])[:, :n_out]


def kernel(state, w1, w2, w3, b, *, block_b=8192):
    if state.ndim == 2:
        state = state[:, None, :]
    B, _, n_in = state.shape
    n_output = 128
    f_p = w1.shape[1]
    out_p = w3.shape[1]

    tb = min(block_b, B)
    while B % tb:
        tb //= 2
    nsteps = B // tb
    grid = (nsteps,)

    flops = 2 * B * (n_in * f_p + f_p * f_p + f_p * out_p)
    bytes_accessed = (
        state.size * state.dtype.itemsize
        + sum(a.size * a.dtype.itemsize for a in (w1, w2, w3, b))
        + B * n_output * 4
    )

    import functools
    body = functools.partial(_mlp_kernel, tb=tb, nsteps=nsteps)

    return pl.pallas_call(
        body,
        out_shape=jax.ShapeDtypeStruct((B, n_output), jnp.float32),
        grid=grid,
        in_specs=[
            pl.BlockSpec(memory_space=pl.ANY),
            pl.BlockSpec(w1.shape, lambda i: (0, 0)),
            pl.BlockSpec(w2.shape, lambda i: (0, 0)),
            pl.BlockSpec(w3.shape, lambda i: (0, 0)),
            pl.BlockSpec(b.shape, lambda i: (0, 0)),
        ],
        out_specs=pl.BlockSpec((tb, n_output), lambda i: (i, 0)),
        scratch_shapes=[
            pltpu.VMEM((2, tb, n_in), jnp.float32),
            pltpu.SemaphoreType.DMA((2,)),
        ],
        compiler_params=pltpu.CompilerParams(
            dimension_semantics=("arbitrary",)),
        cost_estimate=pl.CostEstimate(
            flops=flops, transcendentals=0, bytes_accessed=bytes_accessed),
    )(state, w1, w2, w3, b)


# speedup vs baseline: 1.0189x; 1.0189x over previous
import jax
import jax.numpy as jnp
from jax.experimental import pallas as pl
from jax.experimental.pallas import tpu as pltpu

SUBLANE = 8


def _mlp_kernel(x_hbm, w1_ref, w2_ref, w3_ref, b_ref, o_hbm,
                x_buf, o_buf, in_sem, out_sem, *, tb, nsteps):
    f_p = w1_ref.shape[1]
    out_p = w3_ref.shape[1]
    n_out = o_buf.shape[-1]

    i = pl.program_id(0)
    cur = jax.lax.rem(i, 2)
    nxt = jax.lax.rem(i + 1, 2)

    def start_copy(blk, slot):
        pltpu.make_async_copy(
            x_hbm.at[pl.ds(blk * tb, tb), 0, :], x_buf.at[slot],
            in_sem.at[slot]).start()

    def out_copy(slot, blk):
        return pltpu.make_async_copy(
            o_buf.at[slot], o_hbm.at[pl.ds(blk * tb, tb), :],
            out_sem.at[slot])

    @pl.when(i == 0)
    def _():
        start_copy(0, 0)

    @pl.when(i + 1 < nsteps)
    def _():
        start_copy(i + 1, nxt)

    pltpu.make_async_copy(
        x_hbm.at[pl.ds(0, tb), 0, :], x_buf.at[cur], in_sem.at[cur]).wait()

    @pl.when(i >= 2)
    def _():
        out_copy(cur, 0).wait()

    x = x_buf[cur]
    h = jnp.dot(x, w1_ref[...], preferred_element_type=jnp.float32)
    h = jnp.maximum(h + b_ref[0:1, 0:f_p], 0.0)
    h = jnp.dot(h, w2_ref[...], preferred_element_type=jnp.float32)
    h = jnp.maximum(h + b_ref[1:2, 0:f_p], 0.0)
    a = jnp.dot(h, w3_ref[...], preferred_element_type=jnp.float32)
    o_buf[cur] = (a + b_ref[2:3, 0:out_p])[:, :n_out]
    out_copy(cur, i).start()

    if nsteps > 1:
        @pl.when(i == nsteps - 1)
        def _():
            out_copy(nxt, 0).wait()
            out_copy(cur, 0).wait()
    else:
        out_copy(cur, 0).wait()


def kernel(state, w1, w2, w3, b, *, block_b=4096):
    if state.ndim == 2:
        state = state[:, None, :]
    B, _, n_in = state.shape
    n_output = 128
    f_p = w1.shape[1]
    out_p = w3.shape[1]

    tb = min(block_b, B)
    while B % tb:
        tb //= 2
    nsteps = B // tb
    grid = (nsteps,)

    flops = 2 * B * (n_in * f_p + f_p * f_p + f_p * out_p)
    bytes_accessed = (
        state.size * state.dtype.itemsize
        + sum(a.size * a.dtype.itemsize for a in (w1, w2, w3, b))
        + B * n_output * 4
    )

    import functools
    body = functools.partial(_mlp_kernel, tb=tb, nsteps=nsteps)

    return pl.pallas_call(
        body,
        out_shape=jax.ShapeDtypeStruct((B, n_output), jnp.float32),
        grid=grid,
        in_specs=[
            pl.BlockSpec(memory_space=pl.ANY),
            pl.BlockSpec(w1.shape, lambda i: (0, 0)),
            pl.BlockSpec(w2.shape, lambda i: (0, 0)),
            pl.BlockSpec(w3.shape, lambda i: (0, 0)),
            pl.BlockSpec(b.shape, lambda i: (0, 0)),
        ],
        out_specs=pl.BlockSpec(memory_space=pl.ANY),
        scratch_shapes=[
            pltpu.VMEM((2, tb, n_in), jnp.float32),
            pltpu.VMEM((2, tb, n_output), jnp.float32),
            pltpu.SemaphoreType.DMA((2,)),
            pltpu.SemaphoreType.DMA((2,)),
        ],
        compiler_params=pltpu.CompilerParams(
            dimension_semantics=("arbitrary",)),
        cost_estimate=pl.CostEstimate(
            flops=flops, transcendentals=0, bytes_accessed=bytes_accessed),
    )(state, w1, w2, w3, b)
